# baseline (speedup 1.0000x reference)
import functools

import jax
import jax.numpy as jnp
from jax import lax
from jax.experimental import pallas as pl
from jax.experimental.pallas import tpu as pltpu
from jax.experimental.pallas import tpu_sc as plsc

NN = 10000
NP = 10240
RR = 3
EE = 320000
DD = 128
CH = 80
NT = 16
NCORE = 2

_DEG = RR * NP
_DEG_PER_TILE = _DEG // NT
_A_EDGES_PER_TILE = (RR * EE) // (NCORE * NT)
_A_CHUNKS = _A_EDGES_PER_TILE // CH
_P_CHUNKS_PER_REL = NN // CH


def _stage_a_body(dstoff, permf, x, ones_w, zer1_w, zer2_w,
                  degp, xp,
                  eidx, pidx, ones_v, zb1_v, zb2_v, rows_v, degtbl, sem):
    c = lax.axis_index("c")
    s = lax.axis_index("s")
    wid = c * NT + s
    pltpu.sync_copy(ones_w, ones_v)
    pltpu.sync_copy(zer1_w, zb1_v)
    pltpu.sync_copy(zer2_w, zb2_v)
    pltpu.sync_copy(zb1_v, degtbl.at[pl.ds(s * _DEG_PER_TILE, _DEG_PER_TILE)])
    plsc.subcore_barrier()

    ebase = wid * _A_EDGES_PER_TILE

    def dbody(j, carry):
        pltpu.sync_copy(dstoff.at[pl.ds(ebase + j * CH, CH)], eidx)
        pltpu.sync_copy(ones_v, degtbl.at[eidx], add=True)
        return carry

    lax.fori_loop(0, _A_CHUNKS, dbody, 0)
    plsc.subcore_barrier()
    pltpu.sync_copy(degtbl.at[pl.ds(s * _DEG_PER_TILE, _DEG_PER_TILE)],
                    degp.at[pl.ds(c * _DEG + s * _DEG_PER_TILE, _DEG_PER_TILE)])

    nz = jnp.where(wid < RR, 2, 0)

    def zbody(part, carry):
        pltpu.sync_copy(zb2_v, xp.at[pl.ds(wid * NP + NN + part * 120, 120)])
        return carry

    lax.fori_loop(0, nz, zbody, 0)

    for rel in range(RR):
        nk = (_P_CHUNKS_PER_REL - 1 - wid) // (NCORE * NT) + 1

        def gbody(k, carry):
            j = k * (NCORE * NT) + wid
            pltpu.sync_copy(permf.at[pl.ds(rel * NN + j * CH, CH)], pidx)
            pltpu.async_copy(x.at[pidx], rows_v, sem).wait()
            pltpu.sync_copy(rows_v, xp.at[pl.ds(rel * NP + j * CH, CH)])
            return carry

        lax.fori_loop(0, nk, gbody, 0)


def _stage_a(dstoff, permf, x):
    ones_w = jnp.ones((CH,), jnp.float32)
    zer1_w = jnp.zeros((_DEG_PER_TILE,), jnp.float32)
    zer2_w = jnp.zeros((120, DD), jnp.float32)
    mesh = plsc.VectorSubcoreMesh(core_axis_name="c", subcore_axis_name="s")
    f = functools.partial(
        pl.kernel, mesh=mesh,
        out_type=[jax.ShapeDtypeStruct((NCORE * _DEG,), jnp.float32),
                  jax.ShapeDtypeStruct((RR * NP, DD), jnp.float32)],
        scratch_types=[
            pltpu.VMEM((CH,), jnp.int32),
            pltpu.VMEM((CH,), jnp.int32),
            pltpu.VMEM((CH,), jnp.float32),
            pltpu.VMEM((_DEG_PER_TILE,), jnp.float32),
            pltpu.VMEM((120, DD), jnp.float32),
            pltpu.VMEM((CH, DD), jnp.float32),
            pltpu.VMEM_SHARED((_DEG,), jnp.float32),
            pltpu.SemaphoreType.DMA,
        ],
    )(_stage_a_body)
    return f(dstoff, permf, x, ones_w, zer1_w, zer2_w)


_BB = 8
_NBLK = NP // (_BB * DD)


def _stage_b_kernel(deg_ref, x_ref, xp_ref, w_ref, g_ref):
    deg = deg_ref[0] + deg_ref[1] + 1.0
    dis = lax.rsqrt(deg)[:, :, None]
    w = w_ref[0]
    g_ref[0, 0] = jax.lax.dot_general(
        dis * x_ref[...], w, (((2,), (0,)), ((), ())),
        preferred_element_type=jnp.float32)
    g_ref[1, 0] = jax.lax.dot_general(
        dis * xp_ref[0], w, (((2,), (0,)), ((), ())),
        preferred_element_type=jnp.float32)


def _stage_b(degp, x3, xp3, W):
    return pl.pallas_call(
        _stage_b_kernel,
        grid=(RR, _NBLK),
        in_specs=[
            pl.BlockSpec((NCORE, _BB, DD), lambda r, i: (0, r * _NBLK + i, 0)),
            pl.BlockSpec((_BB, DD, DD), lambda r, i: (i, 0, 0)),
            pl.BlockSpec((1, _BB, DD, DD), lambda r, i: (r, i, 0, 0)),
            pl.BlockSpec((1, DD, DD), lambda r, i: (r, 0, 0)),
        ],
        out_specs=pl.BlockSpec((NCORE, 1, _BB, DD, DD),
                               lambda r, i: (0, r, i, 0, 0)),
        out_shape=jax.ShapeDtypeStruct((NCORE, RR, NP // DD, DD, DD), jnp.float32),
    )(degp, x3, xp3, W)


CHC = 80
_C_RING = 2
_C_EDGES_PER_TILE = 20480
EPAD = _C_EDGES_PER_TILE * NT
_C_CHUNKS = _C_EDGES_PER_TILE // CHC
_ACC_PER_TILE = NP // NT


def _stage_c_body(gflat, srcg, dstf, zeros_w,
                  agg,
                  si_a, si_b, di_a, di_b, rows_a, rows_b,
                  acc, gsem):
    c = lax.axis_index("c")
    s = lax.axis_index("s")
    for rel in range(RR):
        pltpu.sync_copy(zeros_w, rows_a)
        for z in range(_ACC_PER_TILE // CHC):
            pltpu.sync_copy(rows_a, acc.at[pl.ds(s * _ACC_PER_TILE + z * CHC, CHC)])
        plsc.subcore_barrier()

        sbase = (c * RR + rel) * EPAD + s * _C_EDGES_PER_TILE
        dbase = rel * EPAD + s * _C_EDGES_PER_TILE

        def load_idx(q, si, di):
            pltpu.sync_copy(srcg.at[pl.ds(sbase + q * CHC, CHC)], si)
            pltpu.sync_copy(dstf.at[pl.ds(dbase + q * CHC, CHC)], di)

        def gstart(si, rv):
            pltpu.async_copy(gflat.at[si], rv, gsem)

        def gwait(rv):
            pltpu.make_async_copy(gflat.at[si_a], rv, gsem).wait()

        load_idx(0, si_a, di_a)
        gstart(si_a, rows_a)
        load_idx(1, si_b, di_b)

        def cbody(k, carry):
            gwait(rows_a)
            gstart(si_b, rows_b)
            pltpu.sync_copy(rows_a, acc.at[di_a], add=True)
            load_idx(2 * k + 2, si_a, di_a)
            gstart(si_a, rows_a)
            gwait(rows_b)
            pltpu.sync_copy(rows_b, acc.at[di_b], add=True)
            load_idx(2 * k + 3, si_b, di_b)
            return carry

        lax.fori_loop(0, _C_CHUNKS // 2 - 1, cbody, 0)
        gwait(rows_a)
        gstart(si_b, rows_b)
        pltpu.sync_copy(rows_a, acc.at[di_a], add=True)
        gwait(rows_b)
        pltpu.sync_copy(rows_b, acc.at[di_b], add=True)
        plsc.subcore_barrier()
        abase = (c * RR + rel) * NP
        for z in range(5):
            off = s * _ACC_PER_TILE + z * 128
            pltpu.sync_copy(acc.at[pl.ds(off, 128)],
                            agg.at[pl.ds(abase + off, 128)])
        plsc.subcore_barrier()


def _stage_c(gflat, srcg, dstf):
    zeros_w = jnp.zeros((CHC, DD), jnp.float32)
    mesh = plsc.VectorSubcoreMesh(core_axis_name="c", subcore_axis_name="s")
    f = functools.partial(
        pl.kernel, mesh=mesh,
        out_type=jax.ShapeDtypeStruct((NCORE * RR * NP, DD), jnp.float32),
        scratch_types=(
            [pltpu.VMEM((CHC,), jnp.int32)] * 4
            + [pltpu.VMEM((CHC, DD), jnp.float32)] * 2
            + [pltpu.VMEM_SHARED((NP, DD), jnp.float32),
               pltpu.SemaphoreType.DMA]
        ),
    )(_stage_c_body)
    return f(gflat, srcg, dstf, zeros_w)


def _stage_d_kernel(agg_ref, g_ref, deg_ref, b_ref,
                    pos_ref, neg_ref, sum_ref):
    i = pl.program_id(1)
    nblk = pl.num_programs(1)
    deg = deg_ref[0] + deg_ref[1] + 1.0
    dis = lax.rsqrt(deg)[:, :, None]
    bb = b_ref[0, 0][None, None, :]
    p = jnp.maximum(dis * (agg_ref[0, 0] + g_ref[0, 0]) + bb, 0.0)
    q = jnp.maximum(dis * (agg_ref[1, 0] + g_ref[1, 0]) + bb, 0.0)
    pos_ref[0] = p
    neg_ref[0] = q

    a_ids = lax.broadcasted_iota(jnp.int32, (_BB, DD, 1), 0)
    s_ids = lax.broadcasted_iota(jnp.int32, (_BB, DD, 1), 1)
    node = i * (_BB * DD) + a_ids * DD + s_ids
    pm = jnp.where(node < NN, p, 0.0)

    @pl.when(i == 0)
    def _():
        sum_ref[...] = jnp.zeros_like(sum_ref)

    sum_ref[0, 0] += jnp.sum(pm, axis=(0, 1))

    @pl.when(i == nblk - 1)
    def _():
        sum_ref[...] = jax.nn.sigmoid(sum_ref[...] * (1.0 / NN))


def _stage_d(agg5, g5, degp, b):
    return pl.pallas_call(
        _stage_d_kernel,
        grid=(RR, _NBLK),
        in_specs=[
            pl.BlockSpec((NCORE, 1, _BB, DD, DD), lambda r, i: (0, r, i, 0, 0)),
            pl.BlockSpec((NCORE, 1, _BB, DD, DD), lambda r, i: (0, r, i, 0, 0)),
            pl.BlockSpec((NCORE, _BB, DD), lambda r, i: (0, r * _NBLK + i, 0)),
            pl.BlockSpec((1, 1, DD), lambda r, i: (r, 0, 0)),
        ],
        out_specs=[
            pl.BlockSpec((1, _BB, DD, DD), lambda r, i: (r, i, 0, 0)),
            pl.BlockSpec((1, _BB, DD, DD), lambda r, i: (r, i, 0, 0)),
            pl.BlockSpec((1, 1, DD), lambda r, i: (r, 0, 0)),
        ],
        out_shape=[
            jax.ShapeDtypeStruct((RR, NP // DD, DD, DD), jnp.float32),
            jax.ShapeDtypeStruct((RR, NP // DD, DD, DD), jnp.float32),
            jax.ShapeDtypeStruct((RR, 1, DD), jnp.float32),
        ],
    )(agg5, g5, degp, b)


def kernel(x, edge_index, dropout_probability, W, b, perm):
    x = x.astype(jnp.float32)
    ei = edge_index.astype(jnp.int32)
    src = ei[:, 0, :]
    dst = ei[:, 1, :]
    roff = (jnp.arange(RR, dtype=jnp.int32) * NP)[:, None]
    dstoff = (dst + roff).reshape(-1)
    permf = perm.astype(jnp.int32).reshape(-1)

    degp_flat, xp = _stage_a(dstoff, permf, x)
    degp = degp_flat.reshape(NCORE, RR * NP // DD, DD)

    xpad = jnp.concatenate(
        [x, jnp.zeros((NP - NN, DD), jnp.float32)]).reshape(NP // DD, DD, DD)
    xp3 = xp.reshape(RR, NP // DD, DD, DD)
    g5 = _stage_b(degp, xpad, xp3, W.astype(jnp.float32))

    coff = (jnp.arange(NCORE, dtype=jnp.int32) * (RR * NP))[:, None, None]
    convoff = (coff + roff[None]).astype(jnp.int32)
    srcg3 = src[None] + convoff
    pad_src = jnp.broadcast_to(convoff + NN, (NCORE, RR, EPAD - EE))
    srcg = jnp.concatenate([srcg3, pad_src], axis=-1).reshape(-1)
    pad_dst = jnp.full((RR, EPAD - EE), NN, jnp.int32)
    dstf = jnp.concatenate([dst, pad_dst], axis=-1).reshape(-1)
    agg_flat = _stage_c(g5.reshape(NCORE * RR * NP, DD), srcg, dstf)
    agg5 = agg_flat.reshape(NCORE, RR, NP // DD, DD, DD)

    posp, negp, sums = _stage_d(agg5, g5, degp,
                                b.astype(jnp.float32).reshape(RR, 1, DD))
    pos = posp.reshape(RR, NP, DD)[:, :NN]
    neg = negp.reshape(RR, NP, DD)[:, :NN]
    return pos, neg, sums


# speedup vs baseline: 1.6514x; 1.6514x over previous
import functools

import jax
import jax.numpy as jnp
from jax import lax
from jax.experimental import pallas as pl
from jax.experimental.pallas import tpu as pltpu
from jax.experimental.pallas import tpu_sc as plsc

NN = 10000
NP = 10240
RR = 3
EE = 320000
DD = 128
CH = 80
NT = 16
NCORE = 2

_DEG = RR * NP
_DEG_PER_TILE = _DEG // NT
_A_EDGES_PER_TILE = (RR * EE) // (NCORE * NT)
_A_CHUNKS = _A_EDGES_PER_TILE // CH
_P_CHUNKS_PER_REL = NN // CH


def _stage_a_body(dstoff, permf, x, ones_w, zer1_w, zer2_w,
                  degp, xp,
                  eidx, pidx, ones_v, zb1_v, zb2_v, rows_v, degtbl, sem):
    c = lax.axis_index("c")
    s = lax.axis_index("s")
    wid = c * NT + s
    pltpu.sync_copy(ones_w, ones_v)
    pltpu.sync_copy(zer1_w, zb1_v)
    pltpu.sync_copy(zer2_w, zb2_v)
    pltpu.sync_copy(zb1_v, degtbl.at[pl.ds(s * _DEG_PER_TILE, _DEG_PER_TILE)])
    plsc.subcore_barrier()

    ebase = wid * _A_EDGES_PER_TILE

    def dbody(j, carry):
        pltpu.sync_copy(dstoff.at[pl.ds(ebase + j * CH, CH)], eidx)
        pltpu.sync_copy(ones_v, degtbl.at[eidx], add=True)
        return carry

    lax.fori_loop(0, _A_CHUNKS, dbody, 0)
    plsc.subcore_barrier()
    pltpu.sync_copy(degtbl.at[pl.ds(s * _DEG_PER_TILE, _DEG_PER_TILE)],
                    degp.at[pl.ds(c * _DEG + s * _DEG_PER_TILE, _DEG_PER_TILE)])

    nz = jnp.where(wid < RR, 2, 0)

    def zbody(part, carry):
        pltpu.sync_copy(zb2_v, xp.at[pl.ds(wid * NP + NN + part * 120, 120)])
        return carry

    lax.fori_loop(0, nz, zbody, 0)

    for rel in range(RR):
        nk = (_P_CHUNKS_PER_REL - 1 - wid) // (NCORE * NT) + 1

        def gbody(k, carry):
            j = k * (NCORE * NT) + wid
            pltpu.sync_copy(permf.at[pl.ds(rel * NN + j * CH, CH)], pidx)
            pltpu.async_copy(x.at[pidx], rows_v, sem).wait()
            pltpu.sync_copy(rows_v, xp.at[pl.ds(rel * NP + j * CH, CH)])
            return carry

        lax.fori_loop(0, nk, gbody, 0)


def _stage_a(dstoff, permf, x):
    ones_w = jnp.ones((CH,), jnp.float32)
    zer1_w = jnp.zeros((_DEG_PER_TILE,), jnp.float32)
    zer2_w = jnp.zeros((120, DD), jnp.float32)
    mesh = plsc.VectorSubcoreMesh(core_axis_name="c", subcore_axis_name="s")
    f = functools.partial(
        pl.kernel, mesh=mesh,
        out_type=[jax.ShapeDtypeStruct((NCORE * _DEG,), jnp.float32),
                  jax.ShapeDtypeStruct((RR * NP, DD), jnp.float32)],
        scratch_types=[
            pltpu.VMEM((CH,), jnp.int32),
            pltpu.VMEM((CH,), jnp.int32),
            pltpu.VMEM((CH,), jnp.float32),
            pltpu.VMEM((_DEG_PER_TILE,), jnp.float32),
            pltpu.VMEM((120, DD), jnp.float32),
            pltpu.VMEM((CH, DD), jnp.float32),
            pltpu.VMEM_SHARED((_DEG,), jnp.float32),
            pltpu.SemaphoreType.DMA,
        ],
    )(_stage_a_body)
    return f(dstoff, permf, x, ones_w, zer1_w, zer2_w)


_BB = 8
_NBLK = NP // (_BB * DD)


def _stage_b_kernel(deg_ref, x_ref, xp_ref, w_ref, g_ref):
    deg = deg_ref[0] + deg_ref[1] + 1.0
    dis = lax.rsqrt(deg)[:, :, None]
    w = w_ref[0]
    g_ref[0, 0] = jax.lax.dot_general(
        dis * x_ref[...], w, (((2,), (0,)), ((), ())),
        preferred_element_type=jnp.float32)
    g_ref[1, 0] = jax.lax.dot_general(
        dis * xp_ref[0], w, (((2,), (0,)), ((), ())),
        preferred_element_type=jnp.float32)


def _stage_b(degp, x3, xp3, W):
    return pl.pallas_call(
        _stage_b_kernel,
        grid=(RR, _NBLK),
        in_specs=[
            pl.BlockSpec((NCORE, _BB, DD), lambda r, i: (0, r * _NBLK + i, 0)),
            pl.BlockSpec((_BB, DD, DD), lambda r, i: (i, 0, 0)),
            pl.BlockSpec((1, _BB, DD, DD), lambda r, i: (r, i, 0, 0)),
            pl.BlockSpec((1, DD, DD), lambda r, i: (r, 0, 0)),
        ],
        out_specs=pl.BlockSpec((NCORE, 1, _BB, DD, DD),
                               lambda r, i: (0, r, i, 0, 0)),
        out_shape=jax.ShapeDtypeStruct((NCORE, RR, NP // DD, DD, DD), jnp.float32),
    )(degp, x3, xp3, W)


CHC = 80
_C_RING = 2
_C_EDGES_PER_TILE = 20480
EPAD = _C_EDGES_PER_TILE * NT
_C_CHUNKS = _C_EDGES_PER_TILE // CHC
_ACC_PER_TILE = NP // NT


def _stage_c_body(gflat, srcg, dstf, zeros_w,
                  agg,
                  si_a, si_b, di_a, di_b, rows_a, rows_b,
                  acc, gsem):
    c = lax.axis_index("c")
    s = lax.axis_index("s")
    for rel in range(RR):
        pltpu.sync_copy(zeros_w, rows_a)
        for z in range(_ACC_PER_TILE // CHC):
            pltpu.sync_copy(rows_a, acc.at[pl.ds(s * _ACC_PER_TILE + z * CHC, CHC)])
        plsc.subcore_barrier()

        sbase = (c * RR + rel) * EPAD + s * _C_EDGES_PER_TILE
        dbase = rel * EPAD + s * _C_EDGES_PER_TILE

        def load_idx(q, si, di):
            pltpu.sync_copy(srcg.at[pl.ds(sbase + q * CHC, CHC)], si)
            pltpu.sync_copy(dstf.at[pl.ds(dbase + q * CHC, CHC)], di)

        def gstart(si, rv):
            pltpu.async_copy(gflat.at[si], rv, gsem)

        def gwait(rv):
            pltpu.make_async_copy(gflat.at[si_a], rv, gsem).wait()

        load_idx(0, si_a, di_a)
        gstart(si_a, rows_a)
        load_idx(1, si_b, di_b)

        def cbody(k, carry):
            gwait(rows_a)
            gstart(si_b, rows_b)
            pltpu.sync_copy(rows_a, acc.at[di_a], add=True)
            load_idx(2 * k + 2, si_a, di_a)
            gstart(si_a, rows_a)
            gwait(rows_b)
            pltpu.sync_copy(rows_b, acc.at[di_b], add=True)
            load_idx(2 * k + 3, si_b, di_b)
            return carry

        lax.fori_loop(0, _C_CHUNKS // 2 - 1, cbody, 0)
        gwait(rows_a)
        gstart(si_b, rows_b)
        pltpu.sync_copy(rows_a, acc.at[di_a], add=True)
        gwait(rows_b)
        pltpu.sync_copy(rows_b, acc.at[di_b], add=True)
        plsc.subcore_barrier()
        abase = (c * RR + rel) * NP
        for z in range(5):
            off = s * _ACC_PER_TILE + z * 128
            pltpu.sync_copy(acc.at[pl.ds(off, 128)],
                            agg.at[pl.ds(abase + off, 128)])
        plsc.subcore_barrier()


def _stage_c(gflat, srcg, dstf):
    zeros_w = jnp.zeros((CHC, DD), jnp.float32)
    mesh = plsc.VectorSubcoreMesh(core_axis_name="c", subcore_axis_name="s")
    f = functools.partial(
        pl.kernel, mesh=mesh,
        out_type=jax.ShapeDtypeStruct((NCORE * RR * NP, DD), jnp.float32),
        scratch_types=(
            [pltpu.VMEM((CHC,), jnp.int32)] * 4
            + [pltpu.VMEM((CHC, DD), jnp.float32)] * 2
            + [pltpu.VMEM_SHARED((NP, DD), jnp.float32),
               pltpu.SemaphoreType.DMA]
        ),
    )(_stage_c_body)
    return f(gflat, srcg, dstf, zeros_w)


def _stage_d_kernel(agg_ref, g_ref, deg_ref, b_ref,
                    pos_ref, neg_ref, sum_ref):
    i = pl.program_id(1)
    nblk = pl.num_programs(1)
    deg = deg_ref[0] + deg_ref[1] + 1.0
    dis = lax.rsqrt(deg)[:, :, None]
    bb = b_ref[0, 0][None, None, :]
    p = jnp.maximum(dis * (agg_ref[0, 0] + g_ref[0, 0]) + bb, 0.0)
    q = jnp.maximum(dis * (agg_ref[1, 0] + g_ref[1, 0]) + bb, 0.0)
    pos_ref[0] = p
    neg_ref[0] = q

    a_ids = lax.broadcasted_iota(jnp.int32, (_BB, DD, 1), 0)
    s_ids = lax.broadcasted_iota(jnp.int32, (_BB, DD, 1), 1)
    node = i * (_BB * DD) + a_ids * DD + s_ids
    pm = jnp.where(node < NN, p, 0.0)

    @pl.when(i == 0)
    def _():
        sum_ref[...] = jnp.zeros_like(sum_ref)

    sum_ref[0, 0] += jnp.sum(pm, axis=(0, 1))

    @pl.when(i == nblk - 1)
    def _():
        sum_ref[...] = jax.nn.sigmoid(sum_ref[...] * (1.0 / NN))


def _stage_d(agg5, g5, degp, b):
    return pl.pallas_call(
        _stage_d_kernel,
        grid=(RR, _NBLK),
        in_specs=[
            pl.BlockSpec((NCORE, 1, _BB, DD, DD), lambda r, i: (0, r, i, 0, 0)),
            pl.BlockSpec((NCORE, 1, _BB, DD, DD), lambda r, i: (0, r, i, 0, 0)),
            pl.BlockSpec((NCORE, _BB, DD), lambda r, i: (0, r * _NBLK + i, 0)),
            pl.BlockSpec((1, 1, DD), lambda r, i: (r, 0, 0)),
        ],
        out_specs=[
            pl.BlockSpec((1, _BB, DD, DD), lambda r, i: (r, i, 0, 0)),
            pl.BlockSpec((1, _BB, DD, DD), lambda r, i: (r, i, 0, 0)),
            pl.BlockSpec((1, 1, DD), lambda r, i: (r, 0, 0)),
        ],
        out_shape=[
            jax.ShapeDtypeStruct((RR, NP // DD, DD, DD), jnp.float32),
            jax.ShapeDtypeStruct((RR, NP // DD, DD, DD), jnp.float32),
            jax.ShapeDtypeStruct((RR, 1, DD), jnp.float32),
        ],
    )(agg5, g5, degp, b)


def kernel(x, edge_index, dropout_probability, W, b, perm):
    x = x.astype(jnp.float32)
    ei = edge_index.astype(jnp.int32)
    src = ei[:, 0, :]
    dst = ei[:, 1, :]
    roff = (jnp.arange(RR, dtype=jnp.int32) * NP)[:, None]
    dstoff = (dst + roff).reshape(-1)
    permf = perm.astype(jnp.int32).reshape(-1)

    degp_flat, xp = _stage_a(dstoff, permf, x)
    degp = degp_flat.reshape(NCORE, RR * NP // DD, DD)

    xpad = jnp.concatenate(
        [x, jnp.zeros((NP - NN, DD), jnp.float32)]).reshape(NP // DD, DD, DD)
    xp3 = xp.reshape(RR, NP // DD, DD, DD)
    g5 = _stage_b(degp, xpad, xp3, W.astype(jnp.float32))

    coff = (jnp.arange(NCORE, dtype=jnp.int32) * (RR * NP))[:, None, None]
    convoff = (coff + roff[None]).astype(jnp.int32)
    srcg3 = src[None] + convoff
    pad_idx = NN + jnp.arange(EPAD - EE, dtype=jnp.int32) % (NP - NN)
    pad_src = jnp.broadcast_to(convoff + pad_idx, (NCORE, RR, EPAD - EE))
    srcg = jnp.concatenate([srcg3, pad_src], axis=-1).reshape(-1)
    pad_dst = jnp.broadcast_to(pad_idx, (RR, EPAD - EE))
    dstf = jnp.concatenate([dst, pad_dst], axis=-1).reshape(-1)
    agg_flat = _stage_c(g5.reshape(NCORE * RR * NP, DD), srcg, dstf)
    agg5 = agg_flat.reshape(NCORE, RR, NP // DD, DD, DD)

    posp, negp, sums = _stage_d(agg5, g5, degp,
                                b.astype(jnp.float32).reshape(RR, 1, DD))
    pos = posp.reshape(RR, NP, DD)[:, :NN]
    neg = negp.reshape(RR, NP, DD)[:, :NN]
    return pos, neg, sums


# speedup vs baseline: 1.9387x; 1.1740x over previous
import functools

import jax
import jax.numpy as jnp
from jax import lax
from jax.experimental import pallas as pl
from jax.experimental.pallas import tpu as pltpu
from jax.experimental.pallas import tpu_sc as plsc

NN = 10000
NP = 10240
RR = 3
EE = 320000
DD = 128
CH = 80
NT = 16
NCORE = 2

_DEG = RR * NP
_DEG_PER_TILE = _DEG // NT
_A_EDGES_PER_TILE = (RR * EE) // (NCORE * NT)
_A_CHUNKS = _A_EDGES_PER_TILE // CH
_P_CHUNKS_PER_REL = NN // CH


def _stage_a_body(dstoff, permf, x, ones_w, zer1_w, zer2_w,
                  degp, xp,
                  eidx, pidx, ones_v, zb1_v, zb2_v, rows_v, degtbl, sem):
    c = lax.axis_index("c")
    s = lax.axis_index("s")
    wid = c * NT + s
    pltpu.sync_copy(ones_w, ones_v)
    pltpu.sync_copy(zer1_w, zb1_v)
    pltpu.sync_copy(zer2_w, zb2_v)
    pltpu.sync_copy(zb1_v, degtbl.at[pl.ds(s * _DEG_PER_TILE, _DEG_PER_TILE)])
    plsc.subcore_barrier()

    ebase = wid * _A_EDGES_PER_TILE

    def dbody(j, carry):
        pltpu.sync_copy(dstoff.at[pl.ds(ebase + j * CH, CH)], eidx)
        pltpu.sync_copy(ones_v, degtbl.at[eidx], add=True)
        return carry

    lax.fori_loop(0, _A_CHUNKS, dbody, 0)
    plsc.subcore_barrier()
    pltpu.sync_copy(degtbl.at[pl.ds(s * _DEG_PER_TILE, _DEG_PER_TILE)],
                    degp.at[pl.ds(c * _DEG + s * _DEG_PER_TILE, _DEG_PER_TILE)])

    nz = jnp.where(wid < RR, 2, 0)

    def zbody(part, carry):
        pltpu.sync_copy(zb2_v, xp.at[pl.ds(wid * NP + NN + part * 120, 120)])
        return carry

    lax.fori_loop(0, nz, zbody, 0)

    for rel in range(RR):
        nk = (_P_CHUNKS_PER_REL - 1 - wid) // (NCORE * NT) + 1

        def gbody(k, carry):
            j = k * (NCORE * NT) + wid
            pltpu.sync_copy(permf.at[pl.ds(rel * NN + j * CH, CH)], pidx)
            pltpu.async_copy(x.at[pidx], rows_v, sem).wait()
            pltpu.sync_copy(rows_v, xp.at[pl.ds(rel * NP + j * CH, CH)])
            return carry

        lax.fori_loop(0, nk, gbody, 0)


def _stage_a(dstoff, permf, x):
    ones_w = jnp.ones((CH,), jnp.float32)
    zer1_w = jnp.zeros((_DEG_PER_TILE,), jnp.float32)
    zer2_w = jnp.zeros((120, DD), jnp.float32)
    mesh = plsc.VectorSubcoreMesh(core_axis_name="c", subcore_axis_name="s")
    f = functools.partial(
        pl.kernel, mesh=mesh,
        out_type=[jax.ShapeDtypeStruct((NCORE * _DEG,), jnp.float32),
                  jax.ShapeDtypeStruct((RR * NP, DD), jnp.float32)],
        scratch_types=[
            pltpu.VMEM((CH,), jnp.int32),
            pltpu.VMEM((CH,), jnp.int32),
            pltpu.VMEM((CH,), jnp.float32),
            pltpu.VMEM((_DEG_PER_TILE,), jnp.float32),
            pltpu.VMEM((120, DD), jnp.float32),
            pltpu.VMEM((CH, DD), jnp.float32),
            pltpu.VMEM_SHARED((_DEG,), jnp.float32),
            pltpu.SemaphoreType.DMA,
        ],
    )(_stage_a_body)
    return f(dstoff, permf, x, ones_w, zer1_w, zer2_w)


_BB = 8
_NBLK = NP // (_BB * DD)


def _stage_b_kernel(deg_ref, x_ref, xp_ref, w_ref, g_ref):
    deg = deg_ref[0] + deg_ref[1] + 1.0
    dis = lax.rsqrt(deg)[:, :, None]
    w = w_ref[0]
    g_ref[0, 0] = jax.lax.dot_general(
        dis * x_ref[...], w, (((2,), (0,)), ((), ())),
        preferred_element_type=jnp.float32)
    g_ref[1, 0] = jax.lax.dot_general(
        dis * xp_ref[0], w, (((2,), (0,)), ((), ())),
        preferred_element_type=jnp.float32)


def _stage_b(degp, x3, xp3, W):
    return pl.pallas_call(
        _stage_b_kernel,
        grid=(RR, _NBLK),
        in_specs=[
            pl.BlockSpec((NCORE, _BB, DD), lambda r, i: (0, r * _NBLK + i, 0)),
            pl.BlockSpec((_BB, DD, DD), lambda r, i: (i, 0, 0)),
            pl.BlockSpec((1, _BB, DD, DD), lambda r, i: (r, i, 0, 0)),
            pl.BlockSpec((1, DD, DD), lambda r, i: (r, 0, 0)),
        ],
        out_specs=pl.BlockSpec((NCORE, 1, _BB, DD, DD),
                               lambda r, i: (0, r, i, 0, 0)),
        out_shape=jax.ShapeDtypeStruct((NCORE, RR, NP // DD, DD, DD), jnp.float32),
    )(degp, x3, xp3, W)


CHC = 128
_C_RING = 2
_C_EDGES_PER_TILE = 20480
EPAD = _C_EDGES_PER_TILE * NT
_C_CHUNKS = _C_EDGES_PER_TILE // CHC
_ACC_PER_TILE = NP // NT


def _stage_c_body(gflat, srcg, dstf, zeros_w,
                  agg,
                  si_a, si_b, di_a, di_b, rows_a, rows_b,
                  acc, gsem):
    c = lax.axis_index("c")
    s = lax.axis_index("s")
    for rel in range(RR):
        pltpu.sync_copy(zeros_w, rows_a)
        for z in range(_ACC_PER_TILE // CHC):
            pltpu.sync_copy(rows_a, acc.at[pl.ds(s * _ACC_PER_TILE + z * CHC, CHC)])
        plsc.subcore_barrier()

        sbase = (c * RR + rel) * EPAD + s * _C_EDGES_PER_TILE
        dbase = rel * EPAD + s * _C_EDGES_PER_TILE

        def load_idx(q, si, di):
            pltpu.sync_copy(srcg.at[pl.ds(sbase + q * CHC, CHC)], si)
            pltpu.sync_copy(dstf.at[pl.ds(dbase + q * CHC, CHC)], di)

        def gstart(si, rv):
            pltpu.async_copy(gflat.at[si], rv, gsem)

        def gwait(rv):
            pltpu.make_async_copy(gflat.at[si_a], rv, gsem).wait()

        load_idx(0, si_a, di_a)
        gstart(si_a, rows_a)
        load_idx(1, si_b, di_b)

        def cbody(k, carry):
            gwait(rows_a)
            gstart(si_b, rows_b)
            pltpu.sync_copy(rows_a, acc.at[di_a], add=True)
            load_idx(2 * k + 2, si_a, di_a)
            gstart(si_a, rows_a)
            gwait(rows_b)
            pltpu.sync_copy(rows_b, acc.at[di_b], add=True)
            load_idx(2 * k + 3, si_b, di_b)
            return carry

        lax.fori_loop(0, _C_CHUNKS // 2 - 1, cbody, 0)
        gwait(rows_a)
        gstart(si_b, rows_b)
        pltpu.sync_copy(rows_a, acc.at[di_a], add=True)
        gwait(rows_b)
        pltpu.sync_copy(rows_b, acc.at[di_b], add=True)
        plsc.subcore_barrier()
        abase = (c * RR + rel) * NP
        for z in range(5):
            off = s * _ACC_PER_TILE + z * 128
            pltpu.sync_copy(acc.at[pl.ds(off, 128)],
                            agg.at[pl.ds(abase + off, 128)])
        plsc.subcore_barrier()


def _stage_c(gflat, srcg, dstf):
    zeros_w = jnp.zeros((CHC, DD), jnp.float32)
    mesh = plsc.VectorSubcoreMesh(core_axis_name="c", subcore_axis_name="s")
    f = functools.partial(
        pl.kernel, mesh=mesh,
        out_type=jax.ShapeDtypeStruct((NCORE * RR * NP, DD), jnp.float32),
        scratch_types=(
            [pltpu.VMEM((CHC,), jnp.int32)] * 4
            + [pltpu.VMEM((CHC, DD), jnp.float32)] * 2
            + [pltpu.VMEM_SHARED((NP, DD), jnp.float32),
               pltpu.SemaphoreType.DMA]
        ),
    )(_stage_c_body)
    return f(gflat, srcg, dstf, zeros_w)


def _stage_d_kernel(agg_ref, g_ref, deg_ref, b_ref,
                    pos_ref, neg_ref, sum_ref):
    i = pl.program_id(1)
    nblk = pl.num_programs(1)
    deg = deg_ref[0] + deg_ref[1] + 1.0
    dis = lax.rsqrt(deg)[:, :, None]
    bb = b_ref[0, 0][None, None, :]
    p = jnp.maximum(dis * (agg_ref[0, 0] + g_ref[0, 0]) + bb, 0.0)
    q = jnp.maximum(dis * (agg_ref[1, 0] + g_ref[1, 0]) + bb, 0.0)
    pos_ref[0] = p
    neg_ref[0] = q

    a_ids = lax.broadcasted_iota(jnp.int32, (_BB, DD, 1), 0)
    s_ids = lax.broadcasted_iota(jnp.int32, (_BB, DD, 1), 1)
    node = i * (_BB * DD) + a_ids * DD + s_ids
    pm = jnp.where(node < NN, p, 0.0)

    @pl.when(i == 0)
    def _():
        sum_ref[...] = jnp.zeros_like(sum_ref)

    sum_ref[0, 0] += jnp.sum(pm, axis=(0, 1))

    @pl.when(i == nblk - 1)
    def _():
        sum_ref[...] = jax.nn.sigmoid(sum_ref[...] * (1.0 / NN))


def _stage_d(agg5, g5, degp, b):
    return pl.pallas_call(
        _stage_d_kernel,
        grid=(RR, _NBLK),
        in_specs=[
            pl.BlockSpec((NCORE, 1, _BB, DD, DD), lambda r, i: (0, r, i, 0, 0)),
            pl.BlockSpec((NCORE, 1, _BB, DD, DD), lambda r, i: (0, r, i, 0, 0)),
            pl.BlockSpec((NCORE, _BB, DD), lambda r, i: (0, r * _NBLK + i, 0)),
            pl.BlockSpec((1, 1, DD), lambda r, i: (r, 0, 0)),
        ],
        out_specs=[
            pl.BlockSpec((1, _BB, DD, DD), lambda r, i: (r, i, 0, 0)),
            pl.BlockSpec((1, _BB, DD, DD), lambda r, i: (r, i, 0, 0)),
            pl.BlockSpec((1, 1, DD), lambda r, i: (r, 0, 0)),
        ],
        out_shape=[
            jax.ShapeDtypeStruct((RR, NP // DD, DD, DD), jnp.float32),
            jax.ShapeDtypeStruct((RR, NP // DD, DD, DD), jnp.float32),
            jax.ShapeDtypeStruct((RR, 1, DD), jnp.float32),
        ],
    )(agg5, g5, degp, b)


def kernel(x, edge_index, dropout_probability, W, b, perm):
    x = x.astype(jnp.float32)
    ei = edge_index.astype(jnp.int32)
    src = ei[:, 0, :]
    dst = ei[:, 1, :]
    roff = (jnp.arange(RR, dtype=jnp.int32) * NP)[:, None]
    dstoff = (dst + roff).reshape(-1)
    permf = perm.astype(jnp.int32).reshape(-1)

    degp_flat, xp = _stage_a(dstoff, permf, x)
    degp = degp_flat.reshape(NCORE, RR * NP // DD, DD)

    xpad = jnp.concatenate(
        [x, jnp.zeros((NP - NN, DD), jnp.float32)]).reshape(NP // DD, DD, DD)
    xp3 = xp.reshape(RR, NP // DD, DD, DD)
    g5 = _stage_b(degp, xpad, xp3, W.astype(jnp.float32))

    coff = (jnp.arange(NCORE, dtype=jnp.int32) * (RR * NP))[:, None, None]
    convoff = (coff + roff[None]).astype(jnp.int32)
    srcg3 = src[None] + convoff
    pad_idx = NN + jnp.arange(EPAD - EE, dtype=jnp.int32) % (NP - NN)
    pad_src = jnp.broadcast_to(convoff + pad_idx, (NCORE, RR, EPAD - EE))
    srcg = jnp.concatenate([srcg3, pad_src], axis=-1).reshape(-1)
    pad_dst = jnp.broadcast_to(pad_idx, (RR, EPAD - EE))
    dstf = jnp.concatenate([dst, pad_dst], axis=-1).reshape(-1)
    agg_flat = _stage_c(g5.reshape(NCORE * RR * NP, DD), srcg, dstf)
    agg5 = agg_flat.reshape(NCORE, RR, NP // DD, DD, DD)

    posp, negp, sums = _stage_d(agg5, g5, degp,
                                b.astype(jnp.float32).reshape(RR, 1, DD))
    pos = posp.reshape(RR, NP, DD)[:, :NN]
    neg = negp.reshape(RR, NP, DD)[:, :NN]
    return pos, neg, sums


# speedup vs baseline: 2.2333x; 1.1519x over previous
import functools

import jax
import jax.numpy as jnp
from jax import lax
from jax.experimental import pallas as pl
from jax.experimental.pallas import tpu as pltpu
from jax.experimental.pallas import tpu_sc as plsc

NN = 10000
NP = 10240
RR = 3
EE = 320000
DD = 128
CH = 80
NT = 16
NCORE = 2

_DEG = RR * NP
_DEG_PER_TILE = _DEG // NT
_A_EDGES_PER_TILE = (RR * EE) // (NCORE * NT)
_A_CH = 1200
_A_CHUNKS = _A_EDGES_PER_TILE // _A_CH
_P_CHUNKS_PER_REL = NN // CH


def _stage_a_body(dstoff, permf, x, ones_w, zer1_w, zer2_w,
                  degp, xp,
                  eidx, pidx, ones_v, zb1_v, zb2_v, rows_v, degtbl, sem):
    c = lax.axis_index("c")
    s = lax.axis_index("s")
    wid = c * NT + s
    pltpu.sync_copy(ones_w, ones_v)
    pltpu.sync_copy(zer1_w, zb1_v)
    pltpu.sync_copy(zer2_w, zb2_v)
    pltpu.sync_copy(zb1_v, degtbl.at[pl.ds(s * _DEG_PER_TILE, _DEG_PER_TILE)])
    plsc.subcore_barrier()

    ebase = wid * _A_EDGES_PER_TILE

    def dbody(j, carry):
        pltpu.sync_copy(dstoff.at[pl.ds(ebase + j * _A_CH, _A_CH)], eidx)
        pltpu.sync_copy(ones_v, degtbl.at[eidx], add=True)
        return carry

    lax.fori_loop(0, _A_CHUNKS, dbody, 0)
    plsc.subcore_barrier()
    pltpu.sync_copy(degtbl.at[pl.ds(s * _DEG_PER_TILE, _DEG_PER_TILE)],
                    degp.at[pl.ds(c * _DEG + s * _DEG_PER_TILE, _DEG_PER_TILE)])

    nz = jnp.where(wid < RR, 2, 0)

    def zbody(part, carry):
        pltpu.sync_copy(zb2_v, xp.at[pl.ds(wid * NP + NN + part * 120, 120)])
        return carry

    lax.fori_loop(0, nz, zbody, 0)

    for rel in range(RR):
        nk = (_P_CHUNKS_PER_REL - 1 - wid) // (NCORE * NT) + 1

        def gbody(k, carry):
            j = k * (NCORE * NT) + wid
            pltpu.sync_copy(permf.at[pl.ds(rel * NN + j * CH, CH)], pidx)
            pltpu.async_copy(x.at[pidx], rows_v, sem).wait()
            pltpu.sync_copy(rows_v, xp.at[pl.ds(rel * NP + j * CH, CH)])
            return carry

        lax.fori_loop(0, nk, gbody, 0)


def _stage_a(dstoff, permf, x):
    ones_w = jnp.ones((_A_CH,), jnp.float32)
    zer1_w = jnp.zeros((_DEG_PER_TILE,), jnp.float32)
    zer2_w = jnp.zeros((120, DD), jnp.float32)
    mesh = plsc.VectorSubcoreMesh(core_axis_name="c", subcore_axis_name="s")
    f = functools.partial(
        pl.kernel, mesh=mesh,
        out_type=[jax.ShapeDtypeStruct((NCORE * _DEG,), jnp.float32),
                  jax.ShapeDtypeStruct((RR * NP, DD), jnp.float32)],
        scratch_types=[
            pltpu.VMEM((_A_CH,), jnp.int32),
            pltpu.VMEM((CH,), jnp.int32),
            pltpu.VMEM((_A_CH,), jnp.float32),
            pltpu.VMEM((_DEG_PER_TILE,), jnp.float32),
            pltpu.VMEM((120, DD), jnp.float32),
            pltpu.VMEM((CH, DD), jnp.float32),
            pltpu.VMEM_SHARED((_DEG,), jnp.float32),
            pltpu.SemaphoreType.DMA,
        ],
    )(_stage_a_body)
    return f(dstoff, permf, x, ones_w, zer1_w, zer2_w)


_BB = 8
_NBLK = NP // (_BB * DD)


def _stage_b_kernel(deg_ref, x_ref, xp_ref, w_ref, g_ref):
    deg = deg_ref[0] + deg_ref[1] + 1.0
    dis = lax.rsqrt(deg)[:, :, None]
    w = w_ref[0]
    g_ref[0, 0] = jax.lax.dot_general(
        dis * x_ref[...], w, (((2,), (0,)), ((), ())),
        preferred_element_type=jnp.float32)
    g_ref[1, 0] = jax.lax.dot_general(
        dis * xp_ref[0], w, (((2,), (0,)), ((), ())),
        preferred_element_type=jnp.float32)


def _stage_b(degp, x3, xp3, W):
    return pl.pallas_call(
        _stage_b_kernel,
        grid=(RR, _NBLK),
        in_specs=[
            pl.BlockSpec((NCORE, _BB, DD), lambda r, i: (0, r * _NBLK + i, 0)),
            pl.BlockSpec((_BB, DD, DD), lambda r, i: (i, 0, 0)),
            pl.BlockSpec((1, _BB, DD, DD), lambda r, i: (r, i, 0, 0)),
            pl.BlockSpec((1, DD, DD), lambda r, i: (r, 0, 0)),
        ],
        out_specs=pl.BlockSpec((NCORE, 1, _BB, DD, DD),
                               lambda r, i: (0, r, i, 0, 0)),
        out_shape=jax.ShapeDtypeStruct((NCORE, RR, NP // DD, DD, DD), jnp.float32),
    )(degp, x3, xp3, W)


CHC = 128
_C_RING = 2
_C_EDGES_PER_TILE = 20480
EPAD = _C_EDGES_PER_TILE * NT
_C_CHUNKS = _C_EDGES_PER_TILE // CHC
_ACC_PER_TILE = NP // NT


def _stage_c_body(gflat, srcg, dstf, zeros_w,
                  agg,
                  si_a, si_b, di_a, di_b, rows_a, rows_b,
                  acc, gsem):
    c = lax.axis_index("c")
    s = lax.axis_index("s")
    for rel in range(RR):
        pltpu.sync_copy(zeros_w, rows_a)
        for z in range(_ACC_PER_TILE // CHC):
            pltpu.sync_copy(rows_a, acc.at[pl.ds(s * _ACC_PER_TILE + z * CHC, CHC)])
        plsc.subcore_barrier()

        sbase = (c * RR + rel) * EPAD + s * _C_EDGES_PER_TILE
        dbase = rel * EPAD + s * _C_EDGES_PER_TILE

        def load_idx(q, si, di):
            pltpu.sync_copy(srcg.at[pl.ds(sbase + q * CHC, CHC)], si)
            pltpu.sync_copy(dstf.at[pl.ds(dbase + q * CHC, CHC)], di)

        def gstart(si, rv):
            pltpu.async_copy(gflat.at[si], rv, gsem)

        def gwait(rv):
            pltpu.make_async_copy(gflat.at[si_a], rv, gsem).wait()

        load_idx(0, si_a, di_a)
        gstart(si_a, rows_a)
        load_idx(1, si_b, di_b)

        def cbody(k, carry):
            gwait(rows_a)
            gstart(si_b, rows_b)
            pltpu.sync_copy(rows_a, acc.at[di_a], add=True)
            load_idx(2 * k + 2, si_a, di_a)
            gstart(si_a, rows_a)
            gwait(rows_b)
            pltpu.sync_copy(rows_b, acc.at[di_b], add=True)
            load_idx(2 * k + 3, si_b, di_b)
            return carry

        lax.fori_loop(0, _C_CHUNKS // 2 - 1, cbody, 0)
        gwait(rows_a)
        gstart(si_b, rows_b)
        pltpu.sync_copy(rows_a, acc.at[di_a], add=True)
        gwait(rows_b)
        pltpu.sync_copy(rows_b, acc.at[di_b], add=True)
        plsc.subcore_barrier()
        abase = (c * RR + rel) * NP
        for z in range(5):
            off = s * _ACC_PER_TILE + z * 128
            pltpu.sync_copy(acc.at[pl.ds(off, 128)],
                            agg.at[pl.ds(abase + off, 128)])
        plsc.subcore_barrier()


def _stage_c(gflat, srcg, dstf):
    zeros_w = jnp.zeros((CHC, DD), jnp.float32)
    mesh = plsc.VectorSubcoreMesh(core_axis_name="c", subcore_axis_name="s")
    f = functools.partial(
        pl.kernel, mesh=mesh,
        out_type=jax.ShapeDtypeStruct((NCORE * RR * NP, DD), jnp.float32),
        scratch_types=(
            [pltpu.VMEM((CHC,), jnp.int32)] * 4
            + [pltpu.VMEM((CHC, DD), jnp.float32)] * 2
            + [pltpu.VMEM_SHARED((NP, DD), jnp.float32),
               pltpu.SemaphoreType.DMA]
        ),
    )(_stage_c_body)
    return f(gflat, srcg, dstf, zeros_w)


def _stage_d_kernel(agg_ref, g_ref, deg_ref, b_ref,
                    pos_ref, neg_ref, sum_ref):
    i = pl.program_id(1)
    nblk = pl.num_programs(1)
    deg = deg_ref[0] + deg_ref[1] + 1.0
    dis = lax.rsqrt(deg)[:, :, None]
    bb = b_ref[0, 0][None, None, :]
    p = jnp.maximum(dis * (agg_ref[0, 0] + g_ref[0, 0]) + bb, 0.0)
    q = jnp.maximum(dis * (agg_ref[1, 0] + g_ref[1, 0]) + bb, 0.0)
    pos_ref[0] = p
    neg_ref[0] = q

    a_ids = lax.broadcasted_iota(jnp.int32, (_BB, DD, 1), 0)
    s_ids = lax.broadcasted_iota(jnp.int32, (_BB, DD, 1), 1)
    node = i * (_BB * DD) + a_ids * DD + s_ids
    pm = jnp.where(node < NN, p, 0.0)

    @pl.when(i == 0)
    def _():
        sum_ref[...] = jnp.zeros_like(sum_ref)

    sum_ref[0, 0] += jnp.sum(pm, axis=(0, 1))

    @pl.when(i == nblk - 1)
    def _():
        sum_ref[...] = jax.nn.sigmoid(sum_ref[...] * (1.0 / NN))


def _stage_d(agg5, g5, degp, b):
    return pl.pallas_call(
        _stage_d_kernel,
        grid=(RR, _NBLK),
        in_specs=[
            pl.BlockSpec((NCORE, 1, _BB, DD, DD), lambda r, i: (0, r, i, 0, 0)),
            pl.BlockSpec((NCORE, 1, _BB, DD, DD), lambda r, i: (0, r, i, 0, 0)),
            pl.BlockSpec((NCORE, _BB, DD), lambda r, i: (0, r * _NBLK + i, 0)),
            pl.BlockSpec((1, 1, DD), lambda r, i: (r, 0, 0)),
        ],
        out_specs=[
            pl.BlockSpec((1, _BB, DD, DD), lambda r, i: (r, i, 0, 0)),
            pl.BlockSpec((1, _BB, DD, DD), lambda r, i: (r, i, 0, 0)),
            pl.BlockSpec((1, 1, DD), lambda r, i: (r, 0, 0)),
        ],
        out_shape=[
            jax.ShapeDtypeStruct((RR, NP // DD, DD, DD), jnp.float32),
            jax.ShapeDtypeStruct((RR, NP // DD, DD, DD), jnp.float32),
            jax.ShapeDtypeStruct((RR, 1, DD), jnp.float32),
        ],
    )(agg5, g5, degp, b)


def kernel(x, edge_index, dropout_probability, W, b, perm):
    x = x.astype(jnp.float32)
    ei = edge_index.astype(jnp.int32)
    src = ei[:, 0, :]
    dst = ei[:, 1, :]
    roff = (jnp.arange(RR, dtype=jnp.int32) * NP)[:, None]
    dstoff = (dst + roff).reshape(-1)
    permf = perm.astype(jnp.int32).reshape(-1)

    degp_flat, xp = _stage_a(dstoff, permf, x)
    degp = degp_flat.reshape(NCORE, RR * NP // DD, DD)

    xpad = jnp.concatenate(
        [x, jnp.zeros((NP - NN, DD), jnp.float32)]).reshape(NP // DD, DD, DD)
    xp3 = xp.reshape(RR, NP // DD, DD, DD)
    g5 = _stage_b(degp, xpad, xp3, W.astype(jnp.float32))

    coff = (jnp.arange(NCORE, dtype=jnp.int32) * (RR * NP))[:, None, None]
    convoff = (coff + roff[None]).astype(jnp.int32)
    srcg3 = src[None] + convoff
    pad_idx = NN + jnp.arange(EPAD - EE, dtype=jnp.int32) % (NP - NN)
    pad_src = jnp.broadcast_to(convoff + pad_idx, (NCORE, RR, EPAD - EE))
    srcg = jnp.concatenate([srcg3, pad_src], axis=-1).reshape(-1)
    pad_dst = jnp.broadcast_to(pad_idx, (RR, EPAD - EE))
    dstf = jnp.concatenate([dst, pad_dst], axis=-1).reshape(-1)
    agg_flat = _stage_c(g5.reshape(NCORE * RR * NP, DD), srcg, dstf)
    agg5 = agg_flat.reshape(NCORE, RR, NP // DD, DD, DD)

    posp, negp, sums = _stage_d(agg5, g5, degp,
                                b.astype(jnp.float32).reshape(RR, 1, DD))
    pos = posp.reshape(RR, NP, DD)[:, :NN]
    neg = negp.reshape(RR, NP, DD)[:, :NN]
    return pos, neg, sums
